# baseline (speedup 1.0000x reference)
import functools

import jax
import jax.numpy as jnp
from jax import lax
from jax.experimental import pallas as pl
from jax.experimental.pallas import tpu as pltpu
from jax.experimental.pallas import tpu_sc as plsc

NC = 2
NS = 16
NW = NC * NS


def _lin_body(x_ref, w_ref, b_ref, o_ref, *, act):
    y = jnp.dot(x_ref[...], w_ref[...], preferred_element_type=jnp.float32)
    y = y + b_ref[...]
    if act:
        y = jnp.maximum(y, 0.0)
    o_ref[...] = y


def _linear(x, w, b, act=False, blk=2000):
    m, k = x.shape
    _, h = w.shape
    return pl.pallas_call(
        functools.partial(_lin_body, act=act),
        grid=(m // blk,),
        in_specs=[
            pl.BlockSpec((blk, k), lambda i: (i, 0)),
            pl.BlockSpec((k, h), lambda i: (0, 0)),
            pl.BlockSpec((1, h), lambda i: (0, 0)),
        ],
        out_specs=pl.BlockSpec((blk, h), lambda i: (i, 0)),
        out_shape=jax.ShapeDtypeStruct((m, h), jnp.float32),
    )(x, w, b.reshape(1, -1))


def _conv_mm_body(h_ref, agg_ref, w1_ref, b1_ref, w2_ref, b2_ref, o_ref):
    t = h_ref[...] + agg_ref[0] + agg_ref[1]
    c1 = jnp.maximum(
        jnp.dot(t, w1_ref[...], preferred_element_type=jnp.float32) + b1_ref[...], 0.0)
    o_ref[...] = jnp.dot(c1, w2_ref[...], preferred_element_type=jnp.float32) + b2_ref[...]


def _conv_mm(h, agg, w1, b1, w2, b2, blk=2000):
    n, hh = h.shape
    return pl.pallas_call(
        _conv_mm_body,
        grid=(n // blk,),
        in_specs=[
            pl.BlockSpec((blk, hh), lambda i: (i, 0)),
            pl.BlockSpec((2, blk, hh), lambda i: (0, i, 0)),
            pl.BlockSpec((hh, hh), lambda i: (0, 0)),
            pl.BlockSpec((1, hh), lambda i: (0, 0)),
            pl.BlockSpec((hh, hh), lambda i: (0, 0)),
            pl.BlockSpec((1, hh), lambda i: (0, 0)),
        ],
        out_specs=pl.BlockSpec((blk, hh), lambda i: (i, 0)),
        out_shape=jax.ShapeDtypeStruct((n, hh), jnp.float32),
    )(h, agg, w1, b1.reshape(1, -1), w2, b2.reshape(1, -1))


def _bn_ab_body(conv_ref, convf_ref, h_ref, g_ref, bb_ref, wa_ref, wb_ref,
                hn_ref, a_ref, b_ref, *, n):
    cf = convf_ref[...]
    m = jnp.sum(cf, axis=0, keepdims=True) / n
    v = jnp.sum((cf - m) ** 2, axis=0, keepdims=True) / n
    bn = g_ref[...] * (conv_ref[...] - m) / jnp.sqrt(v + 1e-5) + bb_ref[...]
    hn = (h_ref[...] + jnp.maximum(bn, 0.0)) * 0.5
    hn_ref[...] = hn
    a_ref[...] = jnp.dot(hn, wa_ref[...], preferred_element_type=jnp.float32)
    b_ref[...] = jnp.dot(hn, wb_ref[...], preferred_element_type=jnp.float32)


def _bn_ab(conv, h, g, bb, wa, wb, blk=2000):
    n, hh = h.shape
    return pl.pallas_call(
        functools.partial(_bn_ab_body, n=float(n)),
        grid=(n // blk,),
        in_specs=[
            pl.BlockSpec((blk, hh), lambda i: (i, 0)),
            pl.BlockSpec((n, hh), lambda i: (0, 0)),
            pl.BlockSpec((blk, hh), lambda i: (i, 0)),
            pl.BlockSpec((1, hh), lambda i: (0, 0)),
            pl.BlockSpec((1, hh), lambda i: (0, 0)),
            pl.BlockSpec((hh, hh), lambda i: (0, 0)),
            pl.BlockSpec((hh, hh), lambda i: (0, 0)),
        ],
        out_specs=[
            pl.BlockSpec((blk, hh), lambda i: (i, 0)),
            pl.BlockSpec((blk, hh), lambda i: (i, 0)),
            pl.BlockSpec((blk, hh), lambda i: (i, 0)),
        ],
        out_shape=[
            jax.ShapeDtypeStruct((n, hh), jnp.float32),
            jax.ShapeDtypeStruct((n, hh), jnp.float32),
            jax.ShapeDtypeStruct((n, hh), jnp.float32),
        ],
    )(conv, conv, h, g.reshape(1, -1), bb.reshape(1, -1), wa, wb)


def _relu_mm_body(h_ref, w_ref, o_ref):
    o_ref[...] = jnp.dot(jnp.maximum(h_ref[...], 0.0), w_ref[...],
                         preferred_element_type=jnp.float32)


def _relu_mm(h, w, blk=2000):
    n, hh = h.shape
    _, wd = w.shape
    return pl.pallas_call(
        _relu_mm_body,
        grid=(n // blk,),
        in_specs=[
            pl.BlockSpec((blk, hh), lambda i: (i, 0)),
            pl.BlockSpec((hh, wd), lambda i: (0, 0)),
        ],
        out_specs=pl.BlockSpec((blk, wd), lambda i: (i, 0)),
        out_shape=jax.ShapeDtypeStruct((n, wd), jnp.float32),
    )(h, w)


def _edge_upd_body(ea_ref, r_ref, w1c_ref, b1_ref, w2_ref, b2_ref, o_ref):
    ea = ea_ref[...]
    t = jnp.maximum(
        r_ref[...] + jnp.dot(ea, w1c_ref[...], preferred_element_type=jnp.float32)
        + b1_ref[...], 0.0)
    o_ref[...] = ea + (jnp.dot(t, w2_ref[...], preferred_element_type=jnp.float32)
                       + b2_ref[...]) * 0.5


def _edge_upd(ea, r, w1c, b1, w2, b2, blk=2000):
    e, hh = ea.shape
    return pl.pallas_call(
        _edge_upd_body,
        grid=(e // blk,),
        in_specs=[
            pl.BlockSpec((blk, hh), lambda i: (i, 0)),
            pl.BlockSpec((blk, hh), lambda i: (i, 0)),
            pl.BlockSpec((hh, hh), lambda i: (0, 0)),
            pl.BlockSpec((1, hh), lambda i: (0, 0)),
            pl.BlockSpec((hh, hh), lambda i: (0, 0)),
            pl.BlockSpec((1, hh), lambda i: (0, 0)),
        ],
        out_specs=pl.BlockSpec((blk, hh), lambda i: (i, 0)),
        out_shape=jax.ShapeDtypeStruct((e, hh), jnp.float32),
    )(ea, r, w1c, b1.reshape(1, -1), w2, b2.reshape(1, -1))


def _final_body(rpq_ref, ea_ref, w1c_ref, b1_ref, w2_ref, b2_ref, w3_ref, b3_ref,
                o_ref):
    z1 = jnp.maximum(
        rpq_ref[...]
        + jnp.dot(ea_ref[...], w1c_ref[...], preferred_element_type=jnp.float32)
        + b1_ref[...], 0.0)
    z2 = jnp.maximum(
        jnp.dot(z1, w2_ref[...], preferred_element_type=jnp.float32) + b2_ref[...], 0.0)
    o_ref[...] = (jnp.dot(z2, w3_ref[...], preferred_element_type=jnp.float32)
                  + b3_ref[...])


def _final(rpq, ea, w1c, b1, w2, b2, w3, b3, blk=2000):
    e, hh = ea.shape
    wd = rpq.shape[1]
    w2d = w2.shape[1]
    return pl.pallas_call(
        _final_body,
        grid=(e // blk,),
        in_specs=[
            pl.BlockSpec((blk, wd), lambda i: (i, 0)),
            pl.BlockSpec((blk, hh), lambda i: (i, 0)),
            pl.BlockSpec((hh, wd), lambda i: (0, 0)),
            pl.BlockSpec((1, wd), lambda i: (0, 0)),
            pl.BlockSpec((wd, w2d), lambda i: (0, 0)),
            pl.BlockSpec((1, w2d), lambda i: (0, 0)),
            pl.BlockSpec((w2d, 1), lambda i: (0, 0)),
            pl.BlockSpec((1, 1), lambda i: (0, 0)),
        ],
        out_specs=pl.BlockSpec((blk, 1), lambda i: (i, 0)),
        out_shape=jax.ShapeDtypeStruct((e, 1), jnp.float32),
    )(rpq, ea, w1c, b1.reshape(1, -1), w2, b2.reshape(1, -1), w3, b3.reshape(1, 1))


_MESH = plsc.VectorSubcoreMesh(core_axis_name="c", subcore_axis_name="s")

CH = 80


def _tile_rows(e, ch):
    rows = e // ch
    rt = -(-(rows // NW) // 8) * 8
    last = rows - (NW - 1) * rt
    assert 0 < last <= rt and last % 8 == 0
    return rt, last


def _make_msg_agg(n, hh, e, ch):
    rt, last = _tile_rows(e, ch)
    assert rt % 3 == 1 and last % 3 == 1
    rows0 = (n // NS) // 8 * 8
    tail_rows = n - (NS - 1) * rows0
    zrows = 48

    @functools.partial(
        pl.kernel,
        mesh=_MESH,
        out_type=jax.ShapeDtypeStruct((NC, n, hh), jnp.float32),
        scratch_types=[
            pltpu.VMEM((4, ch), jnp.int32),
            pltpu.VMEM((4, ch), jnp.int32),
            pltpu.VMEM((3, ch, hh), jnp.float32),
            pltpu.VMEM((2, ch, hh), jnp.float32),
            pltpu.VMEM((zrows, hh), jnp.float32),
            pltpu.VMEM_SHARED((n, hh), jnp.float32),
            pltpu.SemaphoreType.DMA((4,)),
            pltpu.SemaphoreType.DMA((4,)),
            pltpu.SemaphoreType.DMA((3,)),
            pltpu.SemaphoreType.DMA((2,)),
            pltpu.SemaphoreType.DMA((3,)),
            pltpu.SemaphoreType.DMA,
        ],
    )
    def k(h_hbm, src_hbm, dst_hbm, ea_hbm, out_hbm,
          si_v, di_v, rows_v, ea_v, zero_v, agg_sh,
          sem_is, sem_id, sem_g, sem_e, sem_w, sem_z):
        c = lax.axis_index("c")
        s = lax.axis_index("s")
        wid = c * NS + s
        row_base = wid * rt
        nr = jnp.where(wid == NW - 1, last, rt)
        edge_base = row_base * ch
        zvec = jnp.zeros((16,), jnp.float32)

        def zrow(r, _):
            for j in range(hh // 16):
                zero_v[r, pl.ds(j * 16, 16)] = zvec
            return 0
        lax.fori_loop(0, zrows, zrow, 0)
        zcopies = [pltpu.make_async_copy(
            zero_v, agg_sh.at[pl.ds(s * rows0 + j * zrows, zrows)], sem_z)
            for j in range(rows0 // zrows)]
        for cp in zcopies:
            cp.start()

        @pl.when(s == NS - 1)
        def _():
            pltpu.async_copy(zero_v.at[pl.ds(0, tail_rows - rows0)],
                             agg_sh.at[pl.ds(NS * rows0, tail_rows - rows0)],
                             sem_z)
        for cp in zcopies:
            cp.wait()

        @pl.when(s == NS - 1)
        def _():
            pltpu.make_async_copy(zero_v.at[pl.ds(0, tail_rows - rows0)],
                                  agg_sh.at[pl.ds(NS * rows0,
                                                  tail_rows - rows0)],
                                  sem_z).wait()
        plsc.subcore_barrier()

        def idx_copies(g):
            q = g % 4
            base = edge_base + g * ch
            return (pltpu.make_async_copy(src_hbm.at[pl.ds(base, ch)],
                                          si_v.at[q], sem_is.at[q]),
                    pltpu.make_async_copy(dst_hbm.at[pl.ds(base, ch)],
                                          di_v.at[q], sem_id.at[q]))

        def fire_idx(g):
            for cp in idx_copies(g):
                cp.start()

        def wait_idx(g):
            for cp in idx_copies(g):
                cp.wait()

        def fire_in(g, b):
            pltpu.async_copy(h_hbm.at[si_v.at[g % 4]], rows_v.at[b],
                             sem_g.at[b])
            pltpu.async_copy(ea_hbm.at[pl.ds(edge_base + g * ch, ch)],
                             ea_v.at[g % 2], sem_e.at[g % 2])

        def wait_in(g, b):
            pltpu.make_async_copy(h_hbm.at[si_v.at[g % 4]], rows_v.at[b],
                                  sem_g.at[b]).wait()
            pltpu.make_async_copy(ea_hbm.at[pl.ds(edge_base + g * ch, ch)],
                                  ea_v.at[g % 2], sem_e.at[g % 2]).wait()

        def fire_out(g, b):
            pltpu.async_copy(rows_v.at[b], agg_sh.at[di_v.at[g % 4]],
                             sem_w.at[b], add=True)

        def wait_out(g, b):
            pltpu.make_async_copy(rows_v.at[b], agg_sh.at[di_v.at[g % 4]],
                                  sem_w.at[b]).wait()

        def compute(g, b):
            def body(eb):
                def row(r, _):
                    for j in range(hh // 16):
                        sl = pl.ds(j * 16, 16)
                        rows_v[b, r, sl] = jnp.maximum(
                            rows_v[b, r, sl] + ea_v[eb, r, sl], 0.0)
                    return 0
                lax.fori_loop(0, ch, row, 0)

            @pl.when(g % 2 == 0)
            def _():
                body(0)

            @pl.when(g % 2 == 1)
            def _():
                body(1)

        def step(g, b, first):
            nb = (b + 1) % 3

            @pl.when(g + 1 < nr)
            def _():
                if not first:
                    wait_out(g - 2, nb)
                wait_idx(g + 1)
                fire_in(g + 1, nb)

            @pl.when(g + 2 < nr)
            def _():
                fire_idx(g + 2)
            wait_in(g, b)
            compute(g, b)
            fire_out(g, b)

        fire_idx(0)
        fire_idx(1)
        wait_idx(0)
        fire_in(0, 0)
        step(0, 0, True)
        step(1, 1, True)

        def lbody(i, _):
            g = 2 + i * 3
            step(g, 2, False)
            step(g + 1, 0, False)
            step(g + 2, 1, False)
            return 0
        lax.fori_loop(0, (nr - 2) // 3, lbody, 0)
        step(nr - 2, 2, False)
        step(nr - 1, 0, False)
        wait_out(nr - 3, 1)
        wait_out(nr - 2, 2)
        wait_out(nr - 1, 0)

        plsc.subcore_barrier()
        pltpu.sync_copy(agg_sh.at[pl.ds(s * rows0, rows0)],
                        out_hbm.at[c, pl.ds(s * rows0, rows0)])

        @pl.when(s == NS - 1)
        def _():
            pltpu.sync_copy(agg_sh.at[pl.ds(NS * rows0, tail_rows - rows0)],
                            out_hbm.at[c, pl.ds(NS * rows0, tail_rows - rows0)])

    return k


def _make_gather_pair(n, wd, e, ch):
    rt, last = _tile_rows(e, ch)
    assert rt % 3 == 2 and last % 3 == 2 and (rt - 2) % 3 == 0

    @functools.partial(
        pl.kernel,
        mesh=_MESH,
        out_type=jax.ShapeDtypeStruct((e, wd), jnp.float32),
        scratch_types=[
            pltpu.VMEM((rt, ch), jnp.int32),
            pltpu.VMEM((rt, ch), jnp.int32),
            pltpu.VMEM((3, ch, wd), jnp.float32),
            pltpu.VMEM((3, ch, wd), jnp.float32),
        ] + [pltpu.SemaphoreType.DMA] * 9,
    )
    def k(a_hbm, b_hbm, src_hbm, dst_hbm, out_hbm,
          si_v, di_v, ra_v, rb_v, *sems):
        sem_a = sems[0:3]
        sem_b = sems[3:6]
        sem_w = sems[6:9]
        c = lax.axis_index("c")
        s = lax.axis_index("s")
        wid = c * NS + s
        row_base = wid * rt
        nr = jnp.where(wid == NW - 1, last, rt)
        edge_base = row_base * ch

        @pl.when(wid < NW - 1)
        def _():
            pltpu.sync_copy(src_hbm.at[pl.ds(row_base, rt)], si_v)
            pltpu.sync_copy(dst_hbm.at[pl.ds(row_base, rt)], di_v)

        @pl.when(wid == NW - 1)
        def _():
            pltpu.sync_copy(src_hbm.at[pl.ds(row_base, last)],
                            si_v.at[pl.ds(0, last)])
            pltpu.sync_copy(dst_hbm.at[pl.ds(row_base, last)],
                            di_v.at[pl.ds(0, last)])

        def fire_in(g, b):
            pltpu.async_copy(a_hbm.at[si_v.at[g]], ra_v.at[b], sem_a[b])
            pltpu.async_copy(b_hbm.at[di_v.at[g]], rb_v.at[b], sem_b[b])

        def wait_in(g, b):
            pltpu.make_async_copy(a_hbm.at[si_v.at[g]], ra_v.at[b],
                                  sem_a[b]).wait()
            pltpu.make_async_copy(b_hbm.at[di_v.at[g]], rb_v.at[b],
                                  sem_b[b]).wait()

        def fire_out(g, b):
            pltpu.async_copy(ra_v.at[b],
                             out_hbm.at[pl.ds(edge_base + g * ch, ch)], sem_w[b])

        def wait_out(g, b):
            pltpu.make_async_copy(ra_v.at[b],
                                  out_hbm.at[pl.ds(edge_base + g * ch, ch)],
                                  sem_w[b]).wait()

        def compute(b):
            def row(r, _):
                for j in range(wd // 16):
                    sl = pl.ds(j * 16, 16)
                    ra_v[b, r, sl] = ra_v[b, r, sl] + rb_v[b, r, sl]
                return 0
            lax.fori_loop(0, ch, row, 0)

        def step(g, b, first):
            nb = (b + 1) % 3

            @pl.when(g + 1 < nr)
            def _():
                if not first:
                    wait_out(g - 2, nb)
                fire_in(g + 1, nb)
            wait_in(g, b)
            compute(b)
            fire_out(g, b)

        fire_in(0, 0)
        step(0, 0, True)
        step(1, 1, True)

        def lbody(i, _):
            g = 2 + i * 3
            step(g, 2, False)
            step(g + 1, 0, False)
            step(g + 2, 1, False)
            return 0
        lax.fori_loop(0, (nr - 2) // 3, lbody, 0)
        wait_out(nr - 3, 2)
        wait_out(nr - 2, 0)
        wait_out(nr - 1, 1)

    return k


def _make_gather_pq(n, e, ch, half):
    rt, last = _tile_rows(e, ch)
    assert rt % 3 == 2 and last % 3 == 2 and (rt - 2) % 3 == 0

    @functools.partial(
        pl.kernel,
        mesh=_MESH,
        out_type=jax.ShapeDtypeStruct((e, half), jnp.float32),
        scratch_types=[
            pltpu.VMEM((rt, ch), jnp.int32),
            pltpu.VMEM((rt, ch), jnp.int32),
            pltpu.VMEM((3, ch, 2 * half), jnp.float32),
            pltpu.VMEM((3, ch, 2 * half), jnp.float32),
            pltpu.VMEM((3, ch, half), jnp.float32),
        ] + [pltpu.SemaphoreType.DMA] * 9,
    )
    def k(t_hbm, src_hbm, dst_hbm, out_hbm,
          si_v, di_v, ra_v, rb_v, out_v, *sems):
        sem_a = sems[0:3]
        sem_b = sems[3:6]
        sem_w = sems[6:9]
        c = lax.axis_index("c")
        s = lax.axis_index("s")
        wid = c * NS + s
        row_base = wid * rt
        nr = jnp.where(wid == NW - 1, last, rt)
        edge_base = row_base * ch

        @pl.when(wid < NW - 1)
        def _():
            pltpu.sync_copy(src_hbm.at[pl.ds(row_base, rt)], si_v)
            pltpu.sync_copy(dst_hbm.at[pl.ds(row_base, rt)], di_v)

        @pl.when(wid == NW - 1)
        def _():
            pltpu.sync_copy(src_hbm.at[pl.ds(row_base, last)],
                            si_v.at[pl.ds(0, last)])
            pltpu.sync_copy(dst_hbm.at[pl.ds(row_base, last)],
                            di_v.at[pl.ds(0, last)])

        def fire_in(g, b):
            pltpu.async_copy(t_hbm.at[si_v.at[g]], ra_v.at[b], sem_a[b])
            pltpu.async_copy(t_hbm.at[di_v.at[g]], rb_v.at[b], sem_b[b])

        def wait_in(g, b):
            pltpu.make_async_copy(t_hbm.at[si_v.at[g]], ra_v.at[b],
                                  sem_a[b]).wait()
            pltpu.make_async_copy(t_hbm.at[di_v.at[g]], rb_v.at[b],
                                  sem_b[b]).wait()

        def fire_out(g, b):
            pltpu.async_copy(out_v.at[b],
                             out_hbm.at[pl.ds(edge_base + g * ch, ch)], sem_w[b])

        def wait_out(g, b):
            pltpu.make_async_copy(out_v.at[b],
                                  out_hbm.at[pl.ds(edge_base + g * ch, ch)],
                                  sem_w[b]).wait()

        def compute(b):
            def row(r, _):
                for j in range(half // 16):
                    out_v[b, r, pl.ds(j * 16, 16)] = (
                        ra_v[b, r, pl.ds(j * 16, 16)]
                        + rb_v[b, r, pl.ds(half + j * 16, 16)])
                return 0
            lax.fori_loop(0, ch, row, 0)

        def step(g, b, first):
            nb = (b + 1) % 3

            @pl.when(g + 1 < nr)
            def _():
                if not first:
                    wait_out(g - 2, nb)
                fire_in(g + 1, nb)
            wait_in(g, b)
            compute(b)
            fire_out(g, b)

        fire_in(0, 0)
        step(0, 0, True)
        step(1, 1, True)

        def lbody(i, _):
            g = 2 + i * 3
            step(g, 2, False)
            step(g + 1, 0, False)
            step(g + 2, 1, False)
            return 0
        lax.fori_loop(0, (nr - 2) // 3, lbody, 0)
        wait_out(nr - 3, 2)
        wait_out(nr - 2, 0)
        wait_out(nr - 1, 1)

    return k


def kernel(x, edge_index, edge_attr, params):
    n, d_node = x.shape
    e = edge_index.shape[1]
    hh = params["node_W"].shape[1]

    src1 = edge_index[0]
    dst1 = edge_index[1]
    src = src1.reshape(e // CH, CH)
    dst = dst1.reshape(e // CH, CH)

    msg_agg = _make_msg_agg(n, hh, e, 64)
    gpair_h = _make_gather_pair(n, hh, e, CH)

    w1 = params["mlp_W1"]
    nout = 64
    w1a = jnp.pad(w1[:hh], ((0, 0), (0, nout - 50)))
    w1b = jnp.pad(w1[hh:2 * hh], ((0, 0), (0, nout - 50)))
    w1c = jnp.pad(w1[2 * hh:], ((0, 0), (0, nout - 50)))
    b1 = jnp.pad(params["mlp_b1"], (0, nout - 50))
    w2 = jnp.pad(params["mlp_W2"], ((0, nout - 50), (0, 32 - 25)))
    b2 = jnp.pad(params["mlp_b2"], (0, 32 - 25))
    w3 = jnp.pad(params["mlp_W3"], ((0, 32 - 25), (0, 0)))
    b3 = params["mlp_b3"]
    gpq = _make_gather_pq(n, e, CH, nout)

    h = _linear(x, params["node_W"], params["node_b"])
    ea = _linear(edge_attr, params["edge_W"], params["edge_b"])

    layers = params["layers"]
    rpq = None
    for li, p in enumerate(layers):
        agg = msg_agg(h, src1, dst1, ea)
        conv = _conv_mm(h, agg, p["cW1"], p["cb1"], p["cW2"], p["cb2"])
        h, a_tab, b_tab = _bn_ab(conv, h, p["bn_g"], p["bn_b"],
                                 p["eW1"][:hh], p["eW1"][hh:2 * hh])
        r = gpair_h(a_tab, b_tab, src, dst)
        if li == len(layers) - 1:
            t_tab = _relu_mm(h, jnp.concatenate([w1a, w1b], axis=1))
            rpq = gpq(t_tab, src, dst)
        ea = _edge_upd(ea, r, p["eW1"][2 * hh:], p["eb1"], p["eW2"], p["eb2"])

    logit = _final(rpq, ea, w1c, b1, w2, b2, w3, b3)
    return (h, logit)


# speedup vs baseline: 1.0034x; 1.0034x over previous
import functools

import jax
import jax.numpy as jnp
from jax import lax
from jax.experimental import pallas as pl
from jax.experimental.pallas import tpu as pltpu
from jax.experimental.pallas import tpu_sc as plsc

NC = 2
NS = 16
NW = NC * NS


def _lin_body(x_ref, w_ref, b_ref, o_ref, *, act):
    y = jnp.dot(x_ref[...], w_ref[...], preferred_element_type=jnp.float32)
    y = y + b_ref[...]
    if act:
        y = jnp.maximum(y, 0.0)
    o_ref[...] = y


def _linear(x, w, b, act=False, blk=2000):
    m, k = x.shape
    _, h = w.shape
    return pl.pallas_call(
        functools.partial(_lin_body, act=act),
        grid=(m // blk,),
        in_specs=[
            pl.BlockSpec((blk, k), lambda i: (i, 0)),
            pl.BlockSpec((k, h), lambda i: (0, 0)),
            pl.BlockSpec((1, h), lambda i: (0, 0)),
        ],
        out_specs=pl.BlockSpec((blk, h), lambda i: (i, 0)),
        out_shape=jax.ShapeDtypeStruct((m, h), jnp.float32),
    )(x, w, b.reshape(1, -1))


def _conv_mm_body(h_ref, agg_ref, w1_ref, b1_ref, w2_ref, b2_ref, o_ref):
    t = h_ref[...] + agg_ref[0] + agg_ref[1]
    c1 = jnp.maximum(
        jnp.dot(t, w1_ref[...], preferred_element_type=jnp.float32) + b1_ref[...], 0.0)
    o_ref[...] = jnp.dot(c1, w2_ref[...], preferred_element_type=jnp.float32) + b2_ref[...]


def _conv_mm(h, agg, w1, b1, w2, b2, blk=2000):
    n, hh = h.shape
    return pl.pallas_call(
        _conv_mm_body,
        grid=(n // blk,),
        in_specs=[
            pl.BlockSpec((blk, hh), lambda i: (i, 0)),
            pl.BlockSpec((2, blk, hh), lambda i: (0, i, 0)),
            pl.BlockSpec((hh, hh), lambda i: (0, 0)),
            pl.BlockSpec((1, hh), lambda i: (0, 0)),
            pl.BlockSpec((hh, hh), lambda i: (0, 0)),
            pl.BlockSpec((1, hh), lambda i: (0, 0)),
        ],
        out_specs=pl.BlockSpec((blk, hh), lambda i: (i, 0)),
        out_shape=jax.ShapeDtypeStruct((n, hh), jnp.float32),
    )(h, agg, w1, b1.reshape(1, -1), w2, b2.reshape(1, -1))


def _bn_ab_body(conv_ref, convf_ref, h_ref, g_ref, bb_ref, *rest, n, with_t):
    if with_t:
        wa_ref, wb_ref, wt_ref, hn_ref, a_ref, b_ref, t_ref = rest
    else:
        wa_ref, wb_ref, hn_ref, a_ref, b_ref = rest
    cf = convf_ref[...]
    m = jnp.sum(cf, axis=0, keepdims=True) / n
    v = jnp.sum((cf - m) ** 2, axis=0, keepdims=True) / n
    bn = g_ref[...] * (conv_ref[...] - m) / jnp.sqrt(v + 1e-5) + bb_ref[...]
    hn = (h_ref[...] + jnp.maximum(bn, 0.0)) * 0.5
    hn_ref[...] = hn
    a_ref[...] = jnp.dot(hn, wa_ref[...], preferred_element_type=jnp.float32)
    b_ref[...] = jnp.dot(hn, wb_ref[...], preferred_element_type=jnp.float32)
    if with_t:
        t_ref[...] = jnp.dot(jnp.maximum(hn, 0.0), wt_ref[...],
                             preferred_element_type=jnp.float32)


def _bn_ab(conv, h, g, bb, wa, wb, wt=None, blk=2000):
    n, hh = h.shape
    with_t = wt is not None
    w_ins = [wa, wb] + ([wt] if with_t else [])
    mat_spec = pl.BlockSpec((hh, hh), lambda i: (0, 0))
    out_spec = pl.BlockSpec((blk, hh), lambda i: (i, 0))
    n_out = 4 if with_t else 3
    return pl.pallas_call(
        functools.partial(_bn_ab_body, n=float(n), with_t=with_t),
        grid=(n // blk,),
        in_specs=[
            pl.BlockSpec((blk, hh), lambda i: (i, 0)),
            pl.BlockSpec((n, hh), lambda i: (0, 0)),
            pl.BlockSpec((blk, hh), lambda i: (i, 0)),
            pl.BlockSpec((1, hh), lambda i: (0, 0)),
            pl.BlockSpec((1, hh), lambda i: (0, 0)),
        ] + [mat_spec] * len(w_ins),
        out_specs=[out_spec] * n_out,
        out_shape=[jax.ShapeDtypeStruct((n, hh), jnp.float32)] * n_out,
    )(conv, conv, h, g.reshape(1, -1), bb.reshape(1, -1), *w_ins)


def _relu_mm_body(h_ref, w_ref, o_ref):
    o_ref[...] = jnp.dot(jnp.maximum(h_ref[...], 0.0), w_ref[...],
                         preferred_element_type=jnp.float32)


def _relu_mm(h, w, blk=2000):
    n, hh = h.shape
    _, wd = w.shape
    return pl.pallas_call(
        _relu_mm_body,
        grid=(n // blk,),
        in_specs=[
            pl.BlockSpec((blk, hh), lambda i: (i, 0)),
            pl.BlockSpec((hh, wd), lambda i: (0, 0)),
        ],
        out_specs=pl.BlockSpec((blk, wd), lambda i: (i, 0)),
        out_shape=jax.ShapeDtypeStruct((n, wd), jnp.float32),
    )(h, w)


def _edge_upd_body(ea_ref, r_ref, w1c_ref, b1_ref, w2_ref, b2_ref, o_ref):
    ea = ea_ref[...]
    t = jnp.maximum(
        r_ref[...] + jnp.dot(ea, w1c_ref[...], preferred_element_type=jnp.float32)
        + b1_ref[...], 0.0)
    o_ref[...] = ea + (jnp.dot(t, w2_ref[...], preferred_element_type=jnp.float32)
                       + b2_ref[...]) * 0.5


def _edge_upd(ea, r, w1c, b1, w2, b2, blk=2000):
    e, hh = ea.shape
    return pl.pallas_call(
        _edge_upd_body,
        grid=(e // blk,),
        in_specs=[
            pl.BlockSpec((blk, hh), lambda i: (i, 0)),
            pl.BlockSpec((blk, hh), lambda i: (i, 0)),
            pl.BlockSpec((hh, hh), lambda i: (0, 0)),
            pl.BlockSpec((1, hh), lambda i: (0, 0)),
            pl.BlockSpec((hh, hh), lambda i: (0, 0)),
            pl.BlockSpec((1, hh), lambda i: (0, 0)),
        ],
        out_specs=pl.BlockSpec((blk, hh), lambda i: (i, 0)),
        out_shape=jax.ShapeDtypeStruct((e, hh), jnp.float32),
    )(ea, r, w1c, b1.reshape(1, -1), w2, b2.reshape(1, -1))


def _final_body(rpq_ref, ea_ref, w1c_ref, b1_ref, w2_ref, b2_ref, w3_ref, b3_ref,
                o_ref):
    z1 = jnp.maximum(
        rpq_ref[...]
        + jnp.dot(ea_ref[...], w1c_ref[...], preferred_element_type=jnp.float32)
        + b1_ref[...], 0.0)
    z2 = jnp.maximum(
        jnp.dot(z1, w2_ref[...], preferred_element_type=jnp.float32) + b2_ref[...], 0.0)
    o_ref[...] = (jnp.dot(z2, w3_ref[...], preferred_element_type=jnp.float32)
                  + b3_ref[...])


def _final(rpq, ea, w1c, b1, w2, b2, w3, b3, blk=2000):
    e, hh = ea.shape
    wd = rpq.shape[1]
    w2d = w2.shape[1]
    return pl.pallas_call(
        _final_body,
        grid=(e // blk,),
        in_specs=[
            pl.BlockSpec((blk, wd), lambda i: (i, 0)),
            pl.BlockSpec((blk, hh), lambda i: (i, 0)),
            pl.BlockSpec((hh, wd), lambda i: (0, 0)),
            pl.BlockSpec((1, wd), lambda i: (0, 0)),
            pl.BlockSpec((wd, w2d), lambda i: (0, 0)),
            pl.BlockSpec((1, w2d), lambda i: (0, 0)),
            pl.BlockSpec((w2d, 1), lambda i: (0, 0)),
            pl.BlockSpec((1, 1), lambda i: (0, 0)),
        ],
        out_specs=pl.BlockSpec((blk, 1), lambda i: (i, 0)),
        out_shape=jax.ShapeDtypeStruct((e, 1), jnp.float32),
    )(rpq, ea, w1c, b1.reshape(1, -1), w2, b2.reshape(1, -1), w3, b3.reshape(1, 1))


_MESH = plsc.VectorSubcoreMesh(core_axis_name="c", subcore_axis_name="s")

CH = 80


def _tile_rows(e, ch):
    rows = e // ch
    rt = -(-(rows // NW) // 8) * 8
    last = rows - (NW - 1) * rt
    assert 0 < last <= rt and last % 8 == 0
    return rt, last


def _make_msg_agg(n, hh, e, ch):
    rt, last = _tile_rows(e, ch)
    assert rt % 3 == 1 and last % 3 == 1
    rows0 = (n // NS) // 8 * 8
    tail_rows = n - (NS - 1) * rows0
    zrows = 48

    @functools.partial(
        pl.kernel,
        mesh=_MESH,
        out_type=jax.ShapeDtypeStruct((NC, n, hh), jnp.float32),
        scratch_types=[
            pltpu.VMEM((4, ch), jnp.int32),
            pltpu.VMEM((4, ch), jnp.int32),
            pltpu.VMEM((3, ch, hh), jnp.float32),
            pltpu.VMEM((2, ch, hh), jnp.float32),
            pltpu.VMEM((zrows, hh), jnp.float32),
            pltpu.VMEM_SHARED((n, hh), jnp.float32),
            pltpu.SemaphoreType.DMA((4,)),
            pltpu.SemaphoreType.DMA((4,)),
            pltpu.SemaphoreType.DMA((3,)),
            pltpu.SemaphoreType.DMA((2,)),
            pltpu.SemaphoreType.DMA((3,)),
            pltpu.SemaphoreType.DMA,
        ],
    )
    def k(h_hbm, src_hbm, dst_hbm, ea_hbm, out_hbm,
          si_v, di_v, rows_v, ea_v, zero_v, agg_sh,
          sem_is, sem_id, sem_g, sem_e, sem_w, sem_z):
        c = lax.axis_index("c")
        s = lax.axis_index("s")
        wid = c * NS + s
        row_base = wid * rt
        nr = jnp.where(wid == NW - 1, last, rt)
        edge_base = row_base * ch
        zvec = jnp.zeros((16,), jnp.float32)

        def zrow(r, _):
            for j in range(hh // 16):
                zero_v[r, pl.ds(j * 16, 16)] = zvec
            return 0
        lax.fori_loop(0, zrows, zrow, 0)
        zcopies = [pltpu.make_async_copy(
            zero_v, agg_sh.at[pl.ds(s * rows0 + j * zrows, zrows)], sem_z)
            for j in range(rows0 // zrows)]
        for cp in zcopies:
            cp.start()

        @pl.when(s == NS - 1)
        def _():
            pltpu.async_copy(zero_v.at[pl.ds(0, tail_rows - rows0)],
                             agg_sh.at[pl.ds(NS * rows0, tail_rows - rows0)],
                             sem_z)
        for cp in zcopies:
            cp.wait()

        @pl.when(s == NS - 1)
        def _():
            pltpu.make_async_copy(zero_v.at[pl.ds(0, tail_rows - rows0)],
                                  agg_sh.at[pl.ds(NS * rows0,
                                                  tail_rows - rows0)],
                                  sem_z).wait()
        plsc.subcore_barrier()

        def idx_copies(g):
            q = g % 4
            base = edge_base + g * ch
            return (pltpu.make_async_copy(src_hbm.at[pl.ds(base, ch)],
                                          si_v.at[q], sem_is.at[q]),
                    pltpu.make_async_copy(dst_hbm.at[pl.ds(base, ch)],
                                          di_v.at[q], sem_id.at[q]))

        def fire_idx(g):
            for cp in idx_copies(g):
                cp.start()

        def wait_idx(g):
            for cp in idx_copies(g):
                cp.wait()

        def fire_in(g, b):
            pltpu.async_copy(h_hbm.at[si_v.at[g % 4]], rows_v.at[b],
                             sem_g.at[b])
            pltpu.async_copy(ea_hbm.at[pl.ds(edge_base + g * ch, ch)],
                             ea_v.at[g % 2], sem_e.at[g % 2])

        def wait_in(g, b):
            pltpu.make_async_copy(h_hbm.at[si_v.at[g % 4]], rows_v.at[b],
                                  sem_g.at[b]).wait()
            pltpu.make_async_copy(ea_hbm.at[pl.ds(edge_base + g * ch, ch)],
                                  ea_v.at[g % 2], sem_e.at[g % 2]).wait()

        def fire_out(g, b):
            pltpu.async_copy(rows_v.at[b], agg_sh.at[di_v.at[g % 4]],
                             sem_w.at[b], add=True)

        def wait_out(g, b):
            pltpu.make_async_copy(rows_v.at[b], agg_sh.at[di_v.at[g % 4]],
                                  sem_w.at[b]).wait()

        def compute(g, b):
            def body(eb):
                def row(r2, _):
                    for k in range(2):
                        r = r2 * 2 + k
                        for j in range(hh // 16):
                            sl = pl.ds(j * 16, 16)
                            rows_v[b, r, sl] = jnp.maximum(
                                rows_v[b, r, sl] + ea_v[eb, r, sl], 0.0)
                    return 0
                lax.fori_loop(0, ch // 2, row, 0)

            @pl.when(g % 2 == 0)
            def _():
                body(0)

            @pl.when(g % 2 == 1)
            def _():
                body(1)

        def step(g, b, first):
            nb = (b + 1) % 3

            @pl.when(g + 1 < nr)
            def _():
                if not first:
                    wait_out(g - 2, nb)
                wait_idx(g + 1)
                fire_in(g + 1, nb)

            @pl.when(g + 2 < nr)
            def _():
                fire_idx(g + 2)
            wait_in(g, b)
            compute(g, b)
            fire_out(g, b)

        fire_idx(0)
        fire_idx(1)
        wait_idx(0)
        fire_in(0, 0)
        step(0, 0, True)
        step(1, 1, True)

        def lbody(i, _):
            g = 2 + i * 3
            step(g, 2, False)
            step(g + 1, 0, False)
            step(g + 2, 1, False)
            return 0
        lax.fori_loop(0, (nr - 2) // 3, lbody, 0)
        step(nr - 2, 2, False)
        step(nr - 1, 0, False)
        wait_out(nr - 3, 1)
        wait_out(nr - 2, 2)
        wait_out(nr - 1, 0)

        plsc.subcore_barrier()
        pltpu.sync_copy(agg_sh.at[pl.ds(s * rows0, rows0)],
                        out_hbm.at[c, pl.ds(s * rows0, rows0)])

        @pl.when(s == NS - 1)
        def _():
            pltpu.sync_copy(agg_sh.at[pl.ds(NS * rows0, tail_rows - rows0)],
                            out_hbm.at[c, pl.ds(NS * rows0, tail_rows - rows0)])

    return k


def _make_gather_pair(n, wd, e, ch):
    rt, last = _tile_rows(e, ch)
    assert rt % 3 == 2 and last % 3 == 2 and (rt - 2) % 3 == 0

    @functools.partial(
        pl.kernel,
        mesh=_MESH,
        out_type=jax.ShapeDtypeStruct((e, wd), jnp.float32),
        scratch_types=[
            pltpu.VMEM((rt, ch), jnp.int32),
            pltpu.VMEM((rt, ch), jnp.int32),
            pltpu.VMEM((3, ch, wd), jnp.float32),
            pltpu.VMEM((3, ch, wd), jnp.float32),
        ] + [pltpu.SemaphoreType.DMA] * 9,
    )
    def k(a_hbm, b_hbm, src_hbm, dst_hbm, out_hbm,
          si_v, di_v, ra_v, rb_v, *sems):
        sem_a = sems[0:3]
        sem_b = sems[3:6]
        sem_w = sems[6:9]
        c = lax.axis_index("c")
        s = lax.axis_index("s")
        wid = c * NS + s
        row_base = wid * rt
        nr = jnp.where(wid == NW - 1, last, rt)
        edge_base = row_base * ch

        @pl.when(wid < NW - 1)
        def _():
            pltpu.sync_copy(src_hbm.at[pl.ds(row_base, rt)], si_v)
            pltpu.sync_copy(dst_hbm.at[pl.ds(row_base, rt)], di_v)

        @pl.when(wid == NW - 1)
        def _():
            pltpu.sync_copy(src_hbm.at[pl.ds(row_base, last)],
                            si_v.at[pl.ds(0, last)])
            pltpu.sync_copy(dst_hbm.at[pl.ds(row_base, last)],
                            di_v.at[pl.ds(0, last)])

        def fire_in(g, b):
            pltpu.async_copy(a_hbm.at[si_v.at[g]], ra_v.at[b], sem_a[b])
            pltpu.async_copy(b_hbm.at[di_v.at[g]], rb_v.at[b], sem_b[b])

        def wait_in(g, b):
            pltpu.make_async_copy(a_hbm.at[si_v.at[g]], ra_v.at[b],
                                  sem_a[b]).wait()
            pltpu.make_async_copy(b_hbm.at[di_v.at[g]], rb_v.at[b],
                                  sem_b[b]).wait()

        def fire_out(g, b):
            pltpu.async_copy(ra_v.at[b],
                             out_hbm.at[pl.ds(edge_base + g * ch, ch)], sem_w[b])

        def wait_out(g, b):
            pltpu.make_async_copy(ra_v.at[b],
                                  out_hbm.at[pl.ds(edge_base + g * ch, ch)],
                                  sem_w[b]).wait()

        def compute(b):
            def row(r, _):
                for j in range(wd // 16):
                    sl = pl.ds(j * 16, 16)
                    ra_v[b, r, sl] = ra_v[b, r, sl] + rb_v[b, r, sl]
                return 0
            lax.fori_loop(0, ch, row, 0)

        def step(g, b, first):
            nb = (b + 1) % 3

            @pl.when(g + 1 < nr)
            def _():
                if not first:
                    wait_out(g - 2, nb)
                fire_in(g + 1, nb)
            wait_in(g, b)
            compute(b)
            fire_out(g, b)

        fire_in(0, 0)
        step(0, 0, True)
        step(1, 1, True)

        def lbody(i, _):
            g = 2 + i * 3
            step(g, 2, False)
            step(g + 1, 0, False)
            step(g + 2, 1, False)
            return 0
        lax.fori_loop(0, (nr - 2) // 3, lbody, 0)
        wait_out(nr - 3, 2)
        wait_out(nr - 2, 0)
        wait_out(nr - 1, 1)

    return k


def _make_gather_pq(n, e, ch, half):
    rt, last = _tile_rows(e, ch)
    assert rt % 3 == 2 and last % 3 == 2 and (rt - 2) % 3 == 0

    @functools.partial(
        pl.kernel,
        mesh=_MESH,
        out_type=jax.ShapeDtypeStruct((e, half), jnp.float32),
        scratch_types=[
            pltpu.VMEM((rt, ch), jnp.int32),
            pltpu.VMEM((rt, ch), jnp.int32),
            pltpu.VMEM((3, ch, 2 * half), jnp.float32),
            pltpu.VMEM((3, ch, 2 * half), jnp.float32),
            pltpu.VMEM((3, ch, half), jnp.float32),
        ] + [pltpu.SemaphoreType.DMA] * 9,
    )
    def k(t_hbm, src_hbm, dst_hbm, out_hbm,
          si_v, di_v, ra_v, rb_v, out_v, *sems):
        sem_a = sems[0:3]
        sem_b = sems[3:6]
        sem_w = sems[6:9]
        c = lax.axis_index("c")
        s = lax.axis_index("s")
        wid = c * NS + s
        row_base = wid * rt
        nr = jnp.where(wid == NW - 1, last, rt)
        edge_base = row_base * ch

        @pl.when(wid < NW - 1)
        def _():
            pltpu.sync_copy(src_hbm.at[pl.ds(row_base, rt)], si_v)
            pltpu.sync_copy(dst_hbm.at[pl.ds(row_base, rt)], di_v)

        @pl.when(wid == NW - 1)
        def _():
            pltpu.sync_copy(src_hbm.at[pl.ds(row_base, last)],
                            si_v.at[pl.ds(0, last)])
            pltpu.sync_copy(dst_hbm.at[pl.ds(row_base, last)],
                            di_v.at[pl.ds(0, last)])

        def fire_in(g, b):
            pltpu.async_copy(t_hbm.at[si_v.at[g]], ra_v.at[b], sem_a[b])
            pltpu.async_copy(t_hbm.at[di_v.at[g]], rb_v.at[b], sem_b[b])

        def wait_in(g, b):
            pltpu.make_async_copy(t_hbm.at[si_v.at[g]], ra_v.at[b],
                                  sem_a[b]).wait()
            pltpu.make_async_copy(t_hbm.at[di_v.at[g]], rb_v.at[b],
                                  sem_b[b]).wait()

        def fire_out(g, b):
            pltpu.async_copy(out_v.at[b],
                             out_hbm.at[pl.ds(edge_base + g * ch, ch)], sem_w[b])

        def wait_out(g, b):
            pltpu.make_async_copy(out_v.at[b],
                                  out_hbm.at[pl.ds(edge_base + g * ch, ch)],
                                  sem_w[b]).wait()

        def compute(b):
            def row(r, _):
                for j in range(half // 16):
                    out_v[b, r, pl.ds(j * 16, 16)] = (
                        ra_v[b, r, pl.ds(j * 16, 16)]
                        + rb_v[b, r, pl.ds(half + j * 16, 16)])
                return 0
            lax.fori_loop(0, ch, row, 0)

        def step(g, b, first):
            nb = (b + 1) % 3

            @pl.when(g + 1 < nr)
            def _():
                if not first:
                    wait_out(g - 2, nb)
                fire_in(g + 1, nb)
            wait_in(g, b)
            compute(b)
            fire_out(g, b)

        fire_in(0, 0)
        step(0, 0, True)
        step(1, 1, True)

        def lbody(i, _):
            g = 2 + i * 3
            step(g, 2, False)
            step(g + 1, 0, False)
            step(g + 2, 1, False)
            return 0
        lax.fori_loop(0, (nr - 2) // 3, lbody, 0)
        wait_out(nr - 3, 2)
        wait_out(nr - 2, 0)
        wait_out(nr - 1, 1)

    return k


def kernel(x, edge_index, edge_attr, params):
    n, d_node = x.shape
    e = edge_index.shape[1]
    hh = params["node_W"].shape[1]

    src1 = edge_index[0]
    dst1 = edge_index[1]
    src = src1.reshape(e // CH, CH)
    dst = dst1.reshape(e // CH, CH)

    msg_agg = _make_msg_agg(n, hh, e, 64)
    gpair_h = _make_gather_pair(n, hh, e, CH)

    w1 = params["mlp_W1"]
    nout = 64
    w1a = jnp.pad(w1[:hh], ((0, 0), (0, nout - 50)))
    w1b = jnp.pad(w1[hh:2 * hh], ((0, 0), (0, nout - 50)))
    w1c = jnp.pad(w1[2 * hh:], ((0, 0), (0, nout - 50)))
    b1 = jnp.pad(params["mlp_b1"], (0, nout - 50))
    w2 = jnp.pad(params["mlp_W2"], ((0, nout - 50), (0, 32 - 25)))
    b2 = jnp.pad(params["mlp_b2"], (0, 32 - 25))
    w3 = jnp.pad(params["mlp_W3"], ((0, 32 - 25), (0, 0)))
    b3 = params["mlp_b3"]
    gpq = _make_gather_pq(n, e, CH, nout)

    h = _linear(x, params["node_W"], params["node_b"])
    ea = _linear(edge_attr, params["edge_W"], params["edge_b"])

    layers = params["layers"]
    rpq = None
    for li, p in enumerate(layers):
        last = li == len(layers) - 1
        agg = msg_agg(h, src1, dst1, ea)
        conv = _conv_mm(h, agg, p["cW1"], p["cb1"], p["cW2"], p["cb2"])
        outs = _bn_ab(conv, h, p["bn_g"], p["bn_b"],
                      p["eW1"][:hh], p["eW1"][hh:2 * hh],
                      wt=jnp.concatenate([w1a, w1b], axis=1) if last else None)
        h, a_tab, b_tab = outs[:3]
        r = gpair_h(a_tab, b_tab, src, dst)
        if last:
            rpq = gpq(outs[3], src, dst)
        ea = _edge_upd(ea, r, p["eW1"][2 * hh:], p["eb1"], p["eW2"], p["eb2"])

    logit = _final(rpq, ea, w1c, b1, w2, b2, w3, b3)
    return (h, logit)


# speedup vs baseline: 1.0940x; 1.0902x over previous
import functools

import jax
import jax.numpy as jnp
from jax import lax
from jax.experimental import pallas as pl
from jax.experimental.pallas import tpu as pltpu
from jax.experimental.pallas import tpu_sc as plsc

NC = 2
NS = 16
NW = NC * NS


def _lin_body(x_ref, w_ref, b_ref, o_ref, *, act):
    y = jnp.dot(x_ref[...], w_ref[...], preferred_element_type=jnp.float32)
    y = y + b_ref[...]
    if act:
        y = jnp.maximum(y, 0.0)
    o_ref[...] = y


def _linear(x, w, b, act=False, blk=2000):
    m, k = x.shape
    _, h = w.shape
    return pl.pallas_call(
        functools.partial(_lin_body, act=act),
        grid=(m // blk,),
        in_specs=[
            pl.BlockSpec((blk, k), lambda i: (i, 0)),
            pl.BlockSpec((k, h), lambda i: (0, 0)),
            pl.BlockSpec((1, h), lambda i: (0, 0)),
        ],
        out_specs=pl.BlockSpec((blk, h), lambda i: (i, 0)),
        out_shape=jax.ShapeDtypeStruct((m, h), jnp.float32),
    )(x, w, b.reshape(1, -1))


def _conv_mm_body(h_ref, agg_ref, w1_ref, b1_ref, w2_ref, b2_ref, o_ref):
    t = h_ref[...] + agg_ref[0] + agg_ref[1]
    c1 = jnp.maximum(
        jnp.dot(t, w1_ref[...], preferred_element_type=jnp.float32) + b1_ref[...], 0.0)
    o_ref[...] = jnp.dot(c1, w2_ref[...], preferred_element_type=jnp.float32) + b2_ref[...]


def _conv_mm(h, agg, w1, b1, w2, b2, blk=2000):
    n, hh = h.shape
    return pl.pallas_call(
        _conv_mm_body,
        grid=(n // blk,),
        in_specs=[
            pl.BlockSpec((blk, hh), lambda i: (i, 0)),
            pl.BlockSpec((2, blk, hh), lambda i: (0, i, 0)),
            pl.BlockSpec((hh, hh), lambda i: (0, 0)),
            pl.BlockSpec((1, hh), lambda i: (0, 0)),
            pl.BlockSpec((hh, hh), lambda i: (0, 0)),
            pl.BlockSpec((1, hh), lambda i: (0, 0)),
        ],
        out_specs=pl.BlockSpec((blk, hh), lambda i: (i, 0)),
        out_shape=jax.ShapeDtypeStruct((n, hh), jnp.float32),
    )(h, agg, w1, b1.reshape(1, -1), w2, b2.reshape(1, -1))


def _bn_ab_body(conv_ref, convf_ref, h_ref, g_ref, bb_ref, *rest, n, with_t):
    if with_t:
        wa_ref, wb_ref, wt_ref, hn_ref, a_ref, b_ref, t_ref = rest
    else:
        wa_ref, wb_ref, hn_ref, a_ref, b_ref = rest
    cf = convf_ref[...]
    m = jnp.sum(cf, axis=0, keepdims=True) / n
    v = jnp.sum((cf - m) ** 2, axis=0, keepdims=True) / n
    bn = g_ref[...] * (conv_ref[...] - m) / jnp.sqrt(v + 1e-5) + bb_ref[...]
    hn = (h_ref[...] + jnp.maximum(bn, 0.0)) * 0.5
    hn_ref[...] = hn
    a_ref[...] = jnp.dot(hn, wa_ref[...], preferred_element_type=jnp.float32)
    b_ref[...] = jnp.dot(hn, wb_ref[...], preferred_element_type=jnp.float32)
    if with_t:
        t_ref[...] = jnp.dot(jnp.maximum(hn, 0.0), wt_ref[...],
                             preferred_element_type=jnp.float32)


def _bn_ab(conv, h, g, bb, wa, wb, wt=None, blk=2000):
    n, hh = h.shape
    with_t = wt is not None
    w_ins = [wa, wb] + ([wt] if with_t else [])
    mat_spec = pl.BlockSpec((hh, hh), lambda i: (0, 0))
    out_spec = pl.BlockSpec((blk, hh), lambda i: (i, 0))
    n_out = 4 if with_t else 3
    return pl.pallas_call(
        functools.partial(_bn_ab_body, n=float(n), with_t=with_t),
        grid=(n // blk,),
        in_specs=[
            pl.BlockSpec((blk, hh), lambda i: (i, 0)),
            pl.BlockSpec((n, hh), lambda i: (0, 0)),
            pl.BlockSpec((blk, hh), lambda i: (i, 0)),
            pl.BlockSpec((1, hh), lambda i: (0, 0)),
            pl.BlockSpec((1, hh), lambda i: (0, 0)),
        ] + [mat_spec] * len(w_ins),
        out_specs=[out_spec] * n_out,
        out_shape=[jax.ShapeDtypeStruct((n, hh), jnp.float32)] * n_out,
    )(conv, conv, h, g.reshape(1, -1), bb.reshape(1, -1), *w_ins)


def _relu_mm_body(h_ref, w_ref, o_ref):
    o_ref[...] = jnp.dot(jnp.maximum(h_ref[...], 0.0), w_ref[...],
                         preferred_element_type=jnp.float32)


def _relu_mm(h, w, blk=2000):
    n, hh = h.shape
    _, wd = w.shape
    return pl.pallas_call(
        _relu_mm_body,
        grid=(n // blk,),
        in_specs=[
            pl.BlockSpec((blk, hh), lambda i: (i, 0)),
            pl.BlockSpec((hh, wd), lambda i: (0, 0)),
        ],
        out_specs=pl.BlockSpec((blk, wd), lambda i: (i, 0)),
        out_shape=jax.ShapeDtypeStruct((n, wd), jnp.float32),
    )(h, w)


def _edge_upd_body(ea_ref, r_ref, w1c_ref, b1_ref, w2_ref, b2_ref, o_ref):
    ea = ea_ref[...]
    t = jnp.maximum(
        r_ref[...] + jnp.dot(ea, w1c_ref[...], preferred_element_type=jnp.float32)
        + b1_ref[...], 0.0)
    o_ref[...] = ea + (jnp.dot(t, w2_ref[...], preferred_element_type=jnp.float32)
                       + b2_ref[...]) * 0.5


def _edge_upd(ea, r, w1c, b1, w2, b2, blk=4000):
    e, hh = ea.shape
    return pl.pallas_call(
        _edge_upd_body,
        grid=(e // blk,),
        in_specs=[
            pl.BlockSpec((blk, hh), lambda i: (i, 0)),
            pl.BlockSpec((blk, hh), lambda i: (i, 0)),
            pl.BlockSpec((hh, hh), lambda i: (0, 0)),
            pl.BlockSpec((1, hh), lambda i: (0, 0)),
            pl.BlockSpec((hh, hh), lambda i: (0, 0)),
            pl.BlockSpec((1, hh), lambda i: (0, 0)),
        ],
        out_specs=pl.BlockSpec((blk, hh), lambda i: (i, 0)),
        out_shape=jax.ShapeDtypeStruct((e, hh), jnp.float32),
    )(ea, r, w1c, b1.reshape(1, -1), w2, b2.reshape(1, -1))


def _final_body(rpq_ref, ea_ref, w1c_ref, b1_ref, w2_ref, b2_ref, w3_ref, b3_ref,
                o_ref):
    z1 = jnp.maximum(
        rpq_ref[...]
        + jnp.dot(ea_ref[...], w1c_ref[...], preferred_element_type=jnp.float32)
        + b1_ref[...], 0.0)
    z2 = jnp.maximum(
        jnp.dot(z1, w2_ref[...], preferred_element_type=jnp.float32) + b2_ref[...], 0.0)
    o_ref[...] = (jnp.dot(z2, w3_ref[...], preferred_element_type=jnp.float32)
                  + b3_ref[...])


def _final(rpq, ea, w1c, b1, w2, b2, w3, b3, blk=4000):
    e, hh = ea.shape
    wd = rpq.shape[1]
    w2d = w2.shape[1]
    return pl.pallas_call(
        _final_body,
        grid=(e // blk,),
        in_specs=[
            pl.BlockSpec((blk, wd), lambda i: (i, 0)),
            pl.BlockSpec((blk, hh), lambda i: (i, 0)),
            pl.BlockSpec((hh, wd), lambda i: (0, 0)),
            pl.BlockSpec((1, wd), lambda i: (0, 0)),
            pl.BlockSpec((wd, w2d), lambda i: (0, 0)),
            pl.BlockSpec((1, w2d), lambda i: (0, 0)),
            pl.BlockSpec((w2d, 1), lambda i: (0, 0)),
            pl.BlockSpec((1, 1), lambda i: (0, 0)),
        ],
        out_specs=pl.BlockSpec((blk, 1), lambda i: (i, 0)),
        out_shape=jax.ShapeDtypeStruct((e, 1), jnp.float32),
    )(rpq, ea, w1c, b1.reshape(1, -1), w2, b2.reshape(1, -1), w3, b3.reshape(1, 1))


_MESH = plsc.VectorSubcoreMesh(core_axis_name="c", subcore_axis_name="s")

CH = 80


def _tile_rows(e, ch):
    rows = e // ch
    rt = -(-(rows // NW) // 8) * 8
    last = rows - (NW - 1) * rt
    assert 0 < last <= rt and last % 8 == 0
    return rt, last


def _make_msg_agg(n, hh, e, ch):
    rt, last = _tile_rows(e, ch)
    assert rt % 3 == 1 and last % 3 == 1
    rows0 = (n // NS) // 8 * 8
    tail_rows = n - (NS - 1) * rows0
    zrows = 48

    @functools.partial(
        pl.kernel,
        mesh=_MESH,
        out_type=jax.ShapeDtypeStruct((NC, n, hh), jnp.float32),
        scratch_types=[
            pltpu.VMEM((4, ch), jnp.int32),
            pltpu.VMEM((4, ch), jnp.int32),
            pltpu.VMEM((3, ch, hh), jnp.float32),
            pltpu.VMEM((2, ch, hh), jnp.float32),
            pltpu.VMEM((zrows, hh), jnp.float32),
            pltpu.VMEM_SHARED((n, hh), jnp.float32),
            pltpu.SemaphoreType.DMA((4,)),
            pltpu.SemaphoreType.DMA((4,)),
            pltpu.SemaphoreType.DMA((3,)),
            pltpu.SemaphoreType.DMA((2,)),
            pltpu.SemaphoreType.DMA((3,)),
            pltpu.SemaphoreType.DMA,
        ],
    )
    def k(h_hbm, src_hbm, dst_hbm, ea_hbm, out_hbm,
          si_v, di_v, rows_v, ea_v, zero_v, agg_sh,
          sem_is, sem_id, sem_g, sem_e, sem_w, sem_z):
        c = lax.axis_index("c")
        s = lax.axis_index("s")
        wid = c * NS + s
        row_base = wid * rt
        nr = jnp.where(wid == NW - 1, last, rt)
        edge_base = row_base * ch
        zvec = jnp.zeros((16,), jnp.float32)

        def zrow(r, _):
            for j in range(hh // 16):
                zero_v[r, pl.ds(j * 16, 16)] = zvec
            return 0
        lax.fori_loop(0, zrows, zrow, 0)
        zcopies = [pltpu.make_async_copy(
            zero_v, agg_sh.at[pl.ds(s * rows0 + j * zrows, zrows)], sem_z)
            for j in range(rows0 // zrows)]
        for cp in zcopies:
            cp.start()

        @pl.when(s == NS - 1)
        def _():
            pltpu.async_copy(zero_v.at[pl.ds(0, tail_rows - rows0)],
                             agg_sh.at[pl.ds(NS * rows0, tail_rows - rows0)],
                             sem_z)
        for cp in zcopies:
            cp.wait()

        @pl.when(s == NS - 1)
        def _():
            pltpu.make_async_copy(zero_v.at[pl.ds(0, tail_rows - rows0)],
                                  agg_sh.at[pl.ds(NS * rows0,
                                                  tail_rows - rows0)],
                                  sem_z).wait()
        plsc.subcore_barrier()

        def idx_copies(g):
            q = g % 4
            base = edge_base + g * ch
            return (pltpu.make_async_copy(src_hbm.at[pl.ds(base, ch)],
                                          si_v.at[q], sem_is.at[q]),
                    pltpu.make_async_copy(dst_hbm.at[pl.ds(base, ch)],
                                          di_v.at[q], sem_id.at[q]))

        def fire_idx(g):
            for cp in idx_copies(g):
                cp.start()

        def wait_idx(g):
            for cp in idx_copies(g):
                cp.wait()

        def fire_in(g, b):
            pltpu.async_copy(h_hbm.at[si_v.at[g % 4]], rows_v.at[b],
                             sem_g.at[b])
            pltpu.async_copy(ea_hbm.at[pl.ds(edge_base + g * ch, ch)],
                             ea_v.at[g % 2], sem_e.at[g % 2])

        def wait_in(g, b):
            pltpu.make_async_copy(h_hbm.at[si_v.at[g % 4]], rows_v.at[b],
                                  sem_g.at[b]).wait()
            pltpu.make_async_copy(ea_hbm.at[pl.ds(edge_base + g * ch, ch)],
                                  ea_v.at[g % 2], sem_e.at[g % 2]).wait()

        def fire_out(g, b):
            pltpu.async_copy(rows_v.at[b], agg_sh.at[di_v.at[g % 4]],
                             sem_w.at[b], add=True)

        def wait_out(g, b):
            pltpu.make_async_copy(rows_v.at[b], agg_sh.at[di_v.at[g % 4]],
                                  sem_w.at[b]).wait()

        def compute(g, b):
            def body(eb):
                def row(r2, _):
                    for k in range(2):
                        r = r2 * 2 + k
                        for j in range(hh // 16):
                            sl = pl.ds(j * 16, 16)
                            rows_v[b, r, sl] = jnp.maximum(
                                rows_v[b, r, sl] + ea_v[eb, r, sl], 0.0)
                    return 0
                lax.fori_loop(0, ch // 2, row, 0)

            @pl.when(g % 2 == 0)
            def _():
                body(0)

            @pl.when(g % 2 == 1)
            def _():
                body(1)

        def step(g, b, first):
            nb = (b + 1) % 3

            @pl.when(g + 1 < nr)
            def _():
                if not first:
                    wait_out(g - 2, nb)
                wait_idx(g + 1)
                fire_in(g + 1, nb)

            @pl.when(g + 2 < nr)
            def _():
                fire_idx(g + 2)
            wait_in(g, b)
            compute(g, b)
            fire_out(g, b)

        fire_idx(0)
        fire_idx(1)
        wait_idx(0)
        fire_in(0, 0)
        step(0, 0, True)
        step(1, 1, True)

        def lbody(i, _):
            g = 2 + i * 3
            step(g, 2, False)
            step(g + 1, 0, False)
            step(g + 2, 1, False)
            return 0
        lax.fori_loop(0, (nr - 2) // 3, lbody, 0)
        step(nr - 2, 2, False)
        step(nr - 1, 0, False)
        wait_out(nr - 3, 1)
        wait_out(nr - 2, 2)
        wait_out(nr - 1, 0)

        plsc.subcore_barrier()
        pltpu.sync_copy(agg_sh.at[pl.ds(s * rows0, rows0)],
                        out_hbm.at[c, pl.ds(s * rows0, rows0)])

        @pl.when(s == NS - 1)
        def _():
            pltpu.sync_copy(agg_sh.at[pl.ds(NS * rows0, tail_rows - rows0)],
                            out_hbm.at[c, pl.ds(NS * rows0, tail_rows - rows0)])

    return k


def _make_gather_pair(n, wd, e, ch):
    rt, last = _tile_rows(e, ch)
    assert rt % 3 == 2 and last % 3 == 2 and (rt - 2) % 3 == 0

    @functools.partial(
        pl.kernel,
        mesh=_MESH,
        out_type=jax.ShapeDtypeStruct((e, wd), jnp.float32),
        scratch_types=[
            pltpu.VMEM((rt, ch), jnp.int32),
            pltpu.VMEM((rt, ch), jnp.int32),
            pltpu.VMEM((3, ch, wd), jnp.float32),
            pltpu.VMEM((3, ch, wd), jnp.float32),
        ] + [pltpu.SemaphoreType.DMA] * 9,
    )
    def k(a_hbm, b_hbm, src_hbm, dst_hbm, out_hbm,
          si_v, di_v, ra_v, rb_v, *sems):
        sem_a = sems[0:3]
        sem_b = sems[3:6]
        sem_w = sems[6:9]
        c = lax.axis_index("c")
        s = lax.axis_index("s")
        wid = c * NS + s
        row_base = wid * rt
        nr = jnp.where(wid == NW - 1, last, rt)
        edge_base = row_base * ch

        @pl.when(wid < NW - 1)
        def _():
            pltpu.sync_copy(src_hbm.at[pl.ds(row_base, rt)], si_v)
            pltpu.sync_copy(dst_hbm.at[pl.ds(row_base, rt)], di_v)

        @pl.when(wid == NW - 1)
        def _():
            pltpu.sync_copy(src_hbm.at[pl.ds(row_base, last)],
                            si_v.at[pl.ds(0, last)])
            pltpu.sync_copy(dst_hbm.at[pl.ds(row_base, last)],
                            di_v.at[pl.ds(0, last)])

        def fire_in(g, b):
            pltpu.async_copy(a_hbm.at[si_v.at[g]], ra_v.at[b], sem_a[b])
            pltpu.async_copy(b_hbm.at[di_v.at[g]], rb_v.at[b], sem_b[b])

        def wait_in(g, b):
            pltpu.make_async_copy(a_hbm.at[si_v.at[g]], ra_v.at[b],
                                  sem_a[b]).wait()
            pltpu.make_async_copy(b_hbm.at[di_v.at[g]], rb_v.at[b],
                                  sem_b[b]).wait()

        def fire_out(g, b):
            pltpu.async_copy(ra_v.at[b],
                             out_hbm.at[pl.ds(edge_base + g * ch, ch)], sem_w[b])

        def wait_out(g, b):
            pltpu.make_async_copy(ra_v.at[b],
                                  out_hbm.at[pl.ds(edge_base + g * ch, ch)],
                                  sem_w[b]).wait()

        def compute(b):
            def row(r, _):
                for j in range(wd // 16):
                    sl = pl.ds(j * 16, 16)
                    ra_v[b, r, sl] = ra_v[b, r, sl] + rb_v[b, r, sl]
                return 0
            lax.fori_loop(0, ch, row, 0)

        def step(g, b, first):
            nb = (b + 1) % 3

            @pl.when(g + 1 < nr)
            def _():
                if not first:
                    wait_out(g - 2, nb)
                fire_in(g + 1, nb)
            wait_in(g, b)
            compute(b)
            fire_out(g, b)

        fire_in(0, 0)
        step(0, 0, True)
        step(1, 1, True)

        def lbody(i, _):
            g = 2 + i * 3
            step(g, 2, False)
            step(g + 1, 0, False)
            step(g + 2, 1, False)
            return 0
        lax.fori_loop(0, (nr - 2) // 3, lbody, 0)
        wait_out(nr - 3, 2)
        wait_out(nr - 2, 0)
        wait_out(nr - 1, 1)

    return k


def _make_gather_pq(n, e, ch, half):
    rt, last = _tile_rows(e, ch)
    assert rt % 3 == 2 and last % 3 == 2 and (rt - 2) % 3 == 0

    @functools.partial(
        pl.kernel,
        mesh=_MESH,
        out_type=jax.ShapeDtypeStruct((e, half), jnp.float32),
        scratch_types=[
            pltpu.VMEM((rt, ch), jnp.int32),
            pltpu.VMEM((rt, ch), jnp.int32),
            pltpu.VMEM((3, ch, 2 * half), jnp.float32),
            pltpu.VMEM((3, ch, 2 * half), jnp.float32),
            pltpu.VMEM((3, ch, half), jnp.float32),
        ] + [pltpu.SemaphoreType.DMA] * 9,
    )
    def k(t_hbm, src_hbm, dst_hbm, out_hbm,
          si_v, di_v, ra_v, rb_v, out_v, *sems):
        sem_a = sems[0:3]
        sem_b = sems[3:6]
        sem_w = sems[6:9]
        c = lax.axis_index("c")
        s = lax.axis_index("s")
        wid = c * NS + s
        row_base = wid * rt
        nr = jnp.where(wid == NW - 1, last, rt)
        edge_base = row_base * ch

        @pl.when(wid < NW - 1)
        def _():
            pltpu.sync_copy(src_hbm.at[pl.ds(row_base, rt)], si_v)
            pltpu.sync_copy(dst_hbm.at[pl.ds(row_base, rt)], di_v)

        @pl.when(wid == NW - 1)
        def _():
            pltpu.sync_copy(src_hbm.at[pl.ds(row_base, last)],
                            si_v.at[pl.ds(0, last)])
            pltpu.sync_copy(dst_hbm.at[pl.ds(row_base, last)],
                            di_v.at[pl.ds(0, last)])

        def fire_in(g, b):
            pltpu.async_copy(t_hbm.at[si_v.at[g]], ra_v.at[b], sem_a[b])
            pltpu.async_copy(t_hbm.at[di_v.at[g]], rb_v.at[b], sem_b[b])

        def wait_in(g, b):
            pltpu.make_async_copy(t_hbm.at[si_v.at[g]], ra_v.at[b],
                                  sem_a[b]).wait()
            pltpu.make_async_copy(t_hbm.at[di_v.at[g]], rb_v.at[b],
                                  sem_b[b]).wait()

        def fire_out(g, b):
            pltpu.async_copy(out_v.at[b],
                             out_hbm.at[pl.ds(edge_base + g * ch, ch)], sem_w[b])

        def wait_out(g, b):
            pltpu.make_async_copy(out_v.at[b],
                                  out_hbm.at[pl.ds(edge_base + g * ch, ch)],
                                  sem_w[b]).wait()

        def compute(b):
            def row(r, _):
                for j in range(half // 16):
                    out_v[b, r, pl.ds(j * 16, 16)] = (
                        ra_v[b, r, pl.ds(j * 16, 16)]
                        + rb_v[b, r, pl.ds(half + j * 16, 16)])
                return 0
            lax.fori_loop(0, ch, row, 0)

        def step(g, b, first):
            nb = (b + 1) % 3

            @pl.when(g + 1 < nr)
            def _():
                if not first:
                    wait_out(g - 2, nb)
                fire_in(g + 1, nb)
            wait_in(g, b)
            compute(b)
            fire_out(g, b)

        fire_in(0, 0)
        step(0, 0, True)
        step(1, 1, True)

        def lbody(i, _):
            g = 2 + i * 3
            step(g, 2, False)
            step(g + 1, 0, False)
            step(g + 2, 1, False)
            return 0
        lax.fori_loop(0, (nr - 2) // 3, lbody, 0)
        wait_out(nr - 3, 2)
        wait_out(nr - 2, 0)
        wait_out(nr - 1, 1)

    return k


def kernel(x, edge_index, edge_attr, params):
    n, d_node = x.shape
    e = edge_index.shape[1]
    hh = params["node_W"].shape[1]

    src1 = edge_index[0]
    dst1 = edge_index[1]
    src = src1.reshape(e // CH, CH)
    dst = dst1.reshape(e // CH, CH)

    msg_agg = _make_msg_agg(n, hh, e, 64)
    gpair_h = _make_gather_pair(n, hh, e, CH)

    w1 = params["mlp_W1"]
    nout = 64
    w1a = jnp.pad(w1[:hh], ((0, 0), (0, nout - 50)))
    w1b = jnp.pad(w1[hh:2 * hh], ((0, 0), (0, nout - 50)))
    w1c = jnp.pad(w1[2 * hh:], ((0, 0), (0, nout - 50)))
    b1 = jnp.pad(params["mlp_b1"], (0, nout - 50))
    w2 = jnp.pad(params["mlp_W2"], ((0, nout - 50), (0, 32 - 25)))
    b2 = jnp.pad(params["mlp_b2"], (0, 32 - 25))
    w3 = jnp.pad(params["mlp_W3"], ((0, 32 - 25), (0, 0)))
    b3 = params["mlp_b3"]
    gpq = _make_gather_pq(n, e, CH, nout)

    h = _linear(x, params["node_W"], params["node_b"])
    ea = _linear(edge_attr, params["edge_W"], params["edge_b"], blk=4000)

    layers = params["layers"]
    rpq = None
    for li, p in enumerate(layers):
        last = li == len(layers) - 1
        agg = msg_agg(h, src1, dst1, ea)
        conv = _conv_mm(h, agg, p["cW1"], p["cb1"], p["cW2"], p["cb2"])
        outs = _bn_ab(conv, h, p["bn_g"], p["bn_b"],
                      p["eW1"][:hh], p["eW1"][hh:2 * hh],
                      wt=jnp.concatenate([w1a, w1b], axis=1) if last else None)
        h, a_tab, b_tab = outs[:3]
        r = gpair_h(a_tab, b_tab, src, dst)
        if last:
            rpq = gpq(outs[3], src, dst)
        ea = _edge_upd(ea, r, p["eW1"][2 * hh:], p["eb1"], p["eW2"], p["eb2"])

    logit = _final(rpq, ea, w1c, b1, w2, b2, w3, b3)
    return (h, logit)
